# baseline (speedup 1.0000x reference)
import functools

import jax
import jax.numpy as jnp
from jax import lax
from jax.experimental import pallas as pl
from jax.experimental.pallas import tpu as pltpu
from jax.experimental.pallas import tpu_sc as plsc

B = 16384
IN_FEATURES = 64
ED = 10
HD = 32
CHUNK = 128
NC = 2
NS = 16
NW = NC * NS


PER_W = B // NW


def _make_sc_gather():
    mesh = plsc.VectorSubcoreMesh(
        core_axis_name="c", subcore_axis_name="s", num_cores=NC,
        num_subcores=NS)

    @functools.partial(
        pl.kernel,
        mesh=mesh,
        compiler_params=pltpu.CompilerParams(use_tc_tiling_on_sc=True),
        out_type=[
            jax.ShapeDtypeStruct((B, ED), jnp.float32),
            jax.ShapeDtypeStruct((B, ED), jnp.float32),
        ],
        scratch_types=[
            pltpu.VMEM((PER_W + 16,), jnp.int32),
            pltpu.VMEM((PER_W + 16,), jnp.int32),
            pltpu.SemaphoreType.DMA,
            pltpu.SemaphoreType.DMA,
        ],
    )
    def sc_gather(bidx_hbm, zidx_hbm, btab_hbm, ztab_hbm, be_out, ze_out,
                  bidx_v, zidx_v, bsem, zsem):
        wid = lax.axis_index("s") * NC + lax.axis_index("c")
        base = wid * PER_W
        pltpu.sync_copy(bidx_hbm.at[pl.ds(base, PER_W)],
                        bidx_v.at[pl.ds(0, PER_W)])
        pltpu.sync_copy(zidx_hbm.at[pl.ds(base, PER_W)],
                        zidx_v.at[pl.ds(0, PER_W)])

        def fire(i, carry):
            rb = bidx_v[pl.ds(i, 16)][0]
            pltpu.async_copy(btab_hbm.at[pl.ds(rb, 1)],
                             be_out.at[pl.ds(base + i, 1)], bsem)
            rz = zidx_v[pl.ds(i, 16)][0]
            pltpu.async_copy(ztab_hbm.at[pl.ds(rz, 1)],
                             ze_out.at[pl.ds(base + i, 1)], zsem)
            return carry

        lax.fori_loop(0, PER_W, fire, 0)

        def drain(i, carry):
            pltpu.make_async_copy(btab_hbm.at[pl.ds(0, 1)],
                                  be_out.at[pl.ds(base + i, 1)], bsem).wait()
            pltpu.make_async_copy(ztab_hbm.at[pl.ds(0, 1)],
                                  ze_out.at[pl.ds(base + i, 1)], zsem).wait()
            return carry

        lax.fori_loop(0, PER_W, drain, 0)

    return sc_gather


def _mlp_body(be_ref, ze_ref, x_ref, w1a_ref, w1b_ref, w1c_ref, b1_ref,
              w2_ref, b2_ref, w3_ref, b3_ref, o_ref):
    h = (
        jnp.dot(be_ref[...], w1a_ref[...], preferred_element_type=jnp.float32)
        + jnp.dot(ze_ref[...], w1b_ref[...], preferred_element_type=jnp.float32)
        + jnp.dot(x_ref[...], w1c_ref[...], preferred_element_type=jnp.float32)
        + b1_ref[...]
    )
    h = jnp.maximum(h, 0.0)
    h = jnp.dot(h, w2_ref[...], preferred_element_type=jnp.float32) + b2_ref[...]
    h = jnp.maximum(h, 0.0)
    o_ref[...] = (
        jnp.dot(h, w3_ref[...], preferred_element_type=jnp.float32) + b3_ref[...]
    )


def kernel(brand_tensor, zip_tensor, input_tensor, brand_table, zip_table,
           W1, b1, W2, b2, W3, b3):
    be, ze = _make_sc_gather()(brand_tensor, zip_tensor, brand_table,
                               zip_table)

    w1a = W1[:ED]
    w1b = W1[ED:2 * ED]
    w1c = W1[2 * ED:]
    b1_2d = b1.reshape(1, -1)
    b2_2d = b2.reshape(1, -1)
    b3_2d = b3.reshape(1, -1)

    blk = 2048
    h1 = HD * 2
    out = pl.pallas_call(
        _mlp_body,
        grid=(B // blk,),
        in_specs=[
            pl.BlockSpec((blk, ED), lambda i: (i, 0)),
            pl.BlockSpec((blk, ED), lambda i: (i, 0)),
            pl.BlockSpec((blk, IN_FEATURES), lambda i: (i, 0)),
            pl.BlockSpec((ED, h1), lambda i: (0, 0)),
            pl.BlockSpec((ED, h1), lambda i: (0, 0)),
            pl.BlockSpec((IN_FEATURES, h1), lambda i: (0, 0)),
            pl.BlockSpec((1, h1), lambda i: (0, 0)),
            pl.BlockSpec((h1, HD), lambda i: (0, 0)),
            pl.BlockSpec((1, HD), lambda i: (0, 0)),
            pl.BlockSpec((HD, 1), lambda i: (0, 0)),
            pl.BlockSpec((1, 1), lambda i: (0, 0)),
        ],
        out_specs=pl.BlockSpec((blk, 1), lambda i: (i, 0)),
        out_shape=jax.ShapeDtypeStruct((B, 1), jnp.float32),
    )(be, ze, input_tensor, w1a, w1b, w1c, b1_2d, W2, b2_2d, W3, b3_2d)
    return out
